# baseline (speedup 1.0000x reference)
import jax
import jax.numpy as jnp
from jax import lax
from jax.experimental import pallas as pl
from jax.experimental.pallas import tpu as pltpu
from jax.experimental.pallas import tpu_sc as plsc

F32 = jnp.float32
CH = 80
NB = 8
NB3 = 3
NS = 16
NC = 2
LANES = 16


def _grouped(nch_tile, nb, do_group):
    ngroups, rem = nch_tile // nb, nch_tile % nb

    def group(g, carry):
        do_group(g * nb, nb)
        return carry

    lax.fori_loop(0, ngroups, group, 0)
    if rem:
        do_group(ngroups * nb, rem)


def _stage_idx(idx_v, k, idxb):
    for i in range(CH // LANES):
        idxb[pl.ds(i * LANES, LANES)] = idx_v[k, pl.ds(i * LANES, LANES)]


def _pair_gather(h_node, lidx, ridx):
    E = lidx.shape[0] * lidx.shape[1] * lidx.shape[2]
    D = h_node.shape[1]
    nch_tile = lidx.shape[1]
    mesh = plsc.VectorSubcoreMesh(core_axis_name="c", subcore_axis_name="s")

    def body(hn_hbm, lidx_hbm, ridx_hbm, hl_hbm, hr_hbm, idx_v, buf_v,
             gsem, wsem, *idxb):
        c = lax.axis_index("c")
        s = lax.axis_index("s")

        def run(idx_hbm, out_hbm):
            pltpu.sync_copy(idx_hbm.at[s], idx_v)

            def do_group(k0, nb):
                gds = []
                for b in range(nb):
                    _stage_idx(idx_v, k0 + b, idxb[b])
                    gds.append(pltpu.async_copy(
                        hn_hbm.at[idxb[b]], buf_v.at[b], gsem))
                wds = []
                for b in range(nb):
                    gds[b].wait()
                    wds.append(pltpu.async_copy(
                        buf_v.at[b],
                        out_hbm.at[pl.ds((s * nch_tile + k0 + b) * CH, CH)],
                        wsem))
                for d in wds:
                    d.wait()

            _grouped(nch_tile, NB, do_group)

        @pl.when(c == 0)
        def _():
            run(lidx_hbm, hl_hbm)

        @pl.when(c == 1)
        def _():
            run(ridx_hbm, hr_hbm)

    dt = h_node.dtype
    call = pl.kernel(
        body,
        out_type=[jax.ShapeDtypeStruct((E, D), dt),
                  jax.ShapeDtypeStruct((E, D), dt)],
        mesh=mesh,
        scratch_types=[
            pltpu.VMEM((nch_tile, CH), jnp.int32),
            pltpu.VMEM((NB, CH, D), dt),
            pltpu.SemaphoreType.DMA,
            pltpu.SemaphoreType.DMA,
        ] + [pltpu.VMEM((CH,), jnp.int32) for _ in range(NB)],
    )
    return call(h_node, lidx, ridx)


def _scatter_regather(msgs, idxs, lidx, ridx, p_l, p_r):
    (mlA, mrA, mlB, mrB) = msgs
    (lidxA, ridxA, lidxB, ridxB) = idxs
    E = lidx.shape[0] * lidx.shape[1] * lidx.shape[2]
    D = mlA.shape[1]
    n_pad = p_l.shape[0]
    nch_tile = lidx.shape[1]
    nch_a = lidxA.shape[1]
    nch_b = lidxB.shape[1]
    rows_per_tile = n_pad // NS
    mesh = plsc.VectorSubcoreMesh(core_axis_name="c", subcore_axis_name="s")

    def body(mlA_hbm, mrA_hbm, mlB_hbm, mrB_hbm, lidxA_hbm, ridxA_hbm,
             lidxB_hbm, ridxB_hbm, lidx_hbm, ridx_hbm, pl_hbm, pr_hbm,
             gl_hbm, gr_hbm, accum, idx_v, buf_v, gsem, wsem, *idxb):
        c = lax.axis_index("c")
        s = lax.axis_index("s")

        def init(p_hbm):
            def seed(j, carry):
                r0 = s * rows_per_tile + j * CH
                pltpu.sync_copy(p_hbm.at[pl.ds(r0, CH)], buf_v.at[0])
                pltpu.sync_copy(buf_v.at[0], accum.at[pl.ds(r0, CH)])
                return carry

            lax.fori_loop(0, rows_per_tile // CH, seed, 0)

        def run_scatter(m_hbm, sidx_hbm, nch):
            pltpu.sync_copy(sidx_hbm.at[s], idx_v.at[pl.ds(0, nch)])

            def do_group(k0, nb):
                mds = []
                for b in range(nb):
                    mds.append(pltpu.async_copy(
                        m_hbm.at[pl.ds((s * nch + k0 + b) * CH, CH)],
                        buf_v.at[b], gsem))
                for b in range(nb):
                    _stage_idx(idx_v, k0 + b, idxb[b])
                    mds[b].wait()
                    pltpu.sync_copy(buf_v.at[b], accum.at[idxb[b]], add=True)

            _grouped(nch, NB3, do_group)

        def run_gather(gidx_hbm, out_hbm):
            pltpu.sync_copy(gidx_hbm.at[s], idx_v)

            def do_group(k0, nb):
                wds = []
                for b in range(nb):
                    _stage_idx(idx_v, k0 + b, idxb[b])
                    pltpu.async_copy(accum.at[idxb[b]], buf_v.at[b],
                                     gsem).wait()
                    wds.append(pltpu.async_copy(
                        buf_v.at[b],
                        out_hbm.at[pl.ds((s * nch_tile + k0 + b) * CH, CH)],
                        wsem))
                for d in wds:
                    d.wait()

            _grouped(nch_tile, NB3, do_group)

        @pl.when(c == 0)
        def _():
            init(pl_hbm)

        @pl.when(c == 1)
        def _():
            init(pr_hbm)

        plsc.subcore_barrier()

        @pl.when(c == 0)
        def _():
            run_scatter(mlA_hbm, ridxA_hbm, nch_a)
            run_scatter(mlB_hbm, ridxB_hbm, nch_b)

        @pl.when(c == 1)
        def _():
            run_scatter(mrA_hbm, lidxA_hbm, nch_a)
            run_scatter(mrB_hbm, lidxB_hbm, nch_b)

        plsc.subcore_barrier()

        @pl.when(c == 0)
        def _():
            run_gather(lidx_hbm, gl_hbm)

        @pl.when(c == 1)
        def _():
            run_gather(ridx_hbm, gr_hbm)

    call = pl.kernel(
        body,
        out_type=[jax.ShapeDtypeStruct((E, D), F32),
                  jax.ShapeDtypeStruct((E, D), F32)],
        mesh=mesh,
        scratch_types=[
            pltpu.VMEM_SHARED((n_pad, D), F32),
            pltpu.VMEM((nch_tile, CH), jnp.int32),
            pltpu.VMEM((NB3, CH, D), F32),
            pltpu.SemaphoreType.DMA,
            pltpu.SemaphoreType.DMA,
        ] + [pltpu.VMEM((CH,), jnp.int32) for _ in range(NB3)],
    )
    return call(mlA, mrA, mlB, mrB, lidxA, ridxA, lidxB, ridxB,
                lidx, ridx, p_l, p_r)


def _node_proj(h_node, nl_W, nr_W, n_pad):
    N, D = h_node.shape
    tile_n = n_pad // NS

    def body(hn_ref, nl_ref, nr_ref, pl_ref, pr_ref):
        hn = hn_ref[...]
        pl_ref[...] = jnp.dot(hn, nl_ref[...], preferred_element_type=F32)
        pr_ref[...] = jnp.dot(hn, nr_ref[...], preferred_element_type=F32)

    node_spec = pl.BlockSpec((tile_n, D), lambda i: (i, 0))
    wspec = pl.BlockSpec((D, D), lambda i: (0, 0))
    return pl.pallas_call(
        body,
        grid=(n_pad // tile_n,),
        in_specs=[node_spec, wspec, wspec],
        out_specs=[node_spec, node_spec],
        out_shape=[jax.ShapeDtypeStruct((n_pad, D), F32)] * 2,
    )(h_node, nl_W, nr_W)


def _edge_mlp(xb, xl, xr, t, Wb, Wn, W1, b1, W2, b2, Wgb, Wgn, wgt, bg1,
              Wg2, bg2, *, tile_e, e_off=0):
    E, D = xl.shape
    for cand_t in (tile_e, 1280, 640, 320, 160, E):
        if E % cand_t == 0 and e_off % cand_t == 0:
            tile_e = cand_t
            break
    off_t = e_off // tile_e

    def body(xb_ref, xl_ref, xr_ref, t_ref, Wb_ref, Wn_ref, W1_ref, b1_ref,
             W2_ref, b2_ref, Wgb_ref, Wgn_ref, wgt_ref, bg1_ref, Wg2_ref,
             bg2_ref, al_ref, ar_ref):
        b = xb_ref[...]
        tcol = t_ref[...]
        xn = (xl_ref, xr_ref)
        outs = (al_ref, ar_ref)
        for p in range(2):
            x = xn[p][...].astype(F32)
            inter = (jnp.dot(b, Wb_ref[p], preferred_element_type=F32) *
                     jnp.dot(x, Wn_ref[p], preferred_element_type=F32))
            inter = jnp.maximum(
                jnp.dot(inter, W1_ref[p], preferred_element_type=F32)
                + b1_ref[p], 0.0)
            inter = (jnp.dot(inter, W2_ref[p], preferred_element_type=F32)
                     + b2_ref[p])
            g = (jnp.dot(b, Wgb_ref[p], preferred_element_type=F32)
                 + jnp.dot(x, Wgn_ref[p], preferred_element_type=F32)
                 + tcol * wgt_ref[p] + bg1_ref[p])
            g = (jnp.dot(jnp.maximum(g, 0.0), Wg2_ref[p],
                         preferred_element_type=F32) + bg2_ref[p])
            outs[p][...] = inter * jax.nn.sigmoid(g)

    grid = (E // tile_e,)
    edge_spec = pl.BlockSpec((tile_e, D), lambda i: (i, 0))
    off_spec = pl.BlockSpec((tile_e, D), lambda i: (i + off_t, 0))
    t_spec = pl.BlockSpec((tile_e, 1), lambda i: (i + off_t, 0))

    def wspec(a):
        nd = a.ndim
        return pl.BlockSpec(a.shape, lambda i, _n=nd: (0,) * _n)

    weights = (Wb, Wn, W1, b1, W2, b2, Wgb, Wgn, wgt, bg1, Wg2, bg2)
    return pl.pallas_call(
        body,
        grid=grid,
        in_specs=[off_spec, edge_spec, edge_spec, t_spec]
                 + [wspec(w) for w in weights],
        out_specs=[edge_spec, edge_spec],
        out_shape=[jax.ShapeDtypeStruct((E, D), F32)] * 2,
    )(xb, xl, xr, t, *weights)


def _final(gl, gr, xb, sf_W, dbias, ln_g, ln_b, ot_W, ot_b, *, tile_e):
    E, D = gl.shape

    def body(gl_ref, gr_ref, xb_ref, sfW_ref, dbias_ref, lng_ref, lnb_ref,
             otW_ref, otb_ref, out_ref):
        h = (gl_ref[...] + gr_ref[...]
             + jnp.dot(xb_ref[...], sfW_ref[...], preferred_element_type=F32)
             + dbias_ref[...])
        mu = jnp.mean(h, axis=-1, keepdims=True)
        var = jnp.mean((h - mu) ** 2, axis=-1, keepdims=True)
        h = (h - mu) * lax.rsqrt(var + 1e-5) * lng_ref[...] + lnb_ref[...]
        out_ref[...] = (jnp.dot(jnp.maximum(h, 0.0), otW_ref[...],
                                preferred_element_type=F32) + otb_ref[...])

    grid = (E // tile_e,)
    edge_spec = pl.BlockSpec((tile_e, D), lambda i: (i, 0))

    def wspec(a):
        nd = a.ndim
        return pl.BlockSpec(a.shape, lambda i, _n=nd: (0,) * _n)

    return pl.pallas_call(
        body,
        grid=grid,
        in_specs=[edge_spec, edge_spec, edge_spec, wspec(sf_W), wspec(dbias),
                  wspec(ln_g), wspec(ln_b), wspec(ot_W), wspec(ot_b)],
        out_specs=edge_spec,
        out_shape=jax.ShapeDtypeStruct((E, D), F32),
    )(gl, gr, xb, sf_W, dbias, ln_g, ln_b, ot_W, ot_b)


def kernel(h_bond, bond_index, h_node, bond_time,
           L_Wb, L_Wn, L_W1, L_b1, L_W2, L_b2, L_Wg1, L_bg1, L_Wg2, L_bg2,
           R_Wb, R_Wn, R_W1, R_b1, R_W2, R_b2, R_Wg1, R_bg1, R_Wg2, R_bg2,
           nl_W, nl_b, nr_W, nr_b, sf_W, sf_b, ln_g, ln_b, ot_W, ot_b):
    E, D = h_bond.shape
    N = h_node.shape[0]
    left, right = bond_index[0], bond_index[1]
    lidx = left.reshape(NS, E // CH // NS, CH)
    ridx = right.reshape(NS, E // CH // NS, CH)

    EA = (E * 2 // 5) // (NS * CH) * (NS * CH)
    rsh = lambda a: a.reshape(NS, -1, CH)
    lidxA, ridxA = rsh(left[:EA]), rsh(right[:EA])
    lidxB, ridxB = rsh(left[EA:]), rsh(right[EA:])

    hlA, hrA = _pair_gather(h_node, lidxA, ridxA)
    hlB, hrB = _pair_gather(h_node, lidxB, ridxB)

    n_pad = ((N + NS * CH - 1) // (NS * CH)) * (NS * CH)
    p_l, p_r = _node_proj(h_node, nl_W, nr_W, n_pad)

    stk = lambda a, b: jnp.stack([a, b])
    Wb = stk(L_Wb, R_Wb)
    Wn = stk(L_Wn, R_Wn)
    W1 = stk(L_W1, R_W1)
    b1 = stk(L_b1.reshape(1, -1), R_b1.reshape(1, -1))
    W2 = stk(L_W2, R_W2)
    b2 = stk(L_b2.reshape(1, -1), R_b2.reshape(1, -1))
    Wgb = stk(L_Wg1[:D], R_Wg1[:D])
    Wgn = stk(L_Wg1[D:2 * D], R_Wg1[D:2 * D])
    wgt = stk(L_Wg1[2 * D:], R_Wg1[2 * D:])
    bg1 = stk(L_bg1.reshape(1, -1), R_bg1.reshape(1, -1))
    Wg2 = stk(L_Wg2, R_Wg2)
    bg2 = stk(L_bg2.reshape(1, -1), R_bg2.reshape(1, -1))
    dbias = (nl_b + nr_b + sf_b).reshape(1, D)

    mlA, mrA = _edge_mlp(
        h_bond, hlA, hrA, bond_time, Wb, Wn, W1, b1, W2, b2,
        Wgb, Wgn, wgt, bg1, Wg2, bg2, tile_e=3200)
    mlB, mrB = _edge_mlp(
        h_bond, hlB, hrB, bond_time, Wb, Wn, W1, b1, W2, b2,
        Wgb, Wgn, wgt, bg1, Wg2, bg2, tile_e=3200, e_off=EA)

    gl, gr = _scatter_regather(
        (mlA, mrA, mlB, mrB), (lidxA, ridxA, lidxB, ridxB),
        lidx, ridx, p_l, p_r)

    return _final(gl, gr, h_bond, sf_W, dbias, ln_g.reshape(1, D),
                  ln_b.reshape(1, D), ot_W, ot_b.reshape(1, D), tile_e=3200)


# speedup vs baseline: 1.0089x; 1.0089x over previous
import jax
import jax.numpy as jnp
from jax import lax
from jax.experimental import pallas as pl
from jax.experimental.pallas import tpu as pltpu
from jax.experimental.pallas import tpu_sc as plsc

F32 = jnp.float32
CH = 80
NB = 8
NB3 = 3
NS = 16
NC = 2
LANES = 16


def _grouped(nch_tile, nb, do_group):
    ngroups, rem = nch_tile // nb, nch_tile % nb

    def group(g, carry):
        do_group(g * nb, nb)
        return carry

    lax.fori_loop(0, ngroups, group, 0)
    if rem:
        do_group(ngroups * nb, rem)


def _stage_idx(idx_v, k, idxb):
    for i in range(CH // LANES):
        idxb[pl.ds(i * LANES, LANES)] = idx_v[k, pl.ds(i * LANES, LANES)]


def _pair_gather(h_node, lidx, ridx):
    E = lidx.shape[0] * lidx.shape[1] * lidx.shape[2]
    D = h_node.shape[1]
    nch_tile = lidx.shape[1]
    mesh = plsc.VectorSubcoreMesh(core_axis_name="c", subcore_axis_name="s")

    def body(hn_hbm, lidx_hbm, ridx_hbm, hl_hbm, hr_hbm, idx_v, buf_v,
             gsem, wsem, *idxb):
        c = lax.axis_index("c")
        s = lax.axis_index("s")

        def run(idx_hbm, out_hbm):
            pltpu.sync_copy(idx_hbm.at[s], idx_v)

            def do_group(k0, nb):
                gds = []
                for b in range(nb):
                    _stage_idx(idx_v, k0 + b, idxb[b])
                    gds.append(pltpu.async_copy(
                        hn_hbm.at[idxb[b]], buf_v.at[b], gsem))
                wds = []
                for b in range(nb):
                    gds[b].wait()
                    wds.append(pltpu.async_copy(
                        buf_v.at[b],
                        out_hbm.at[pl.ds((s * nch_tile + k0 + b) * CH, CH)],
                        wsem))
                for d in wds:
                    d.wait()

            _grouped(nch_tile, NB, do_group)

        @pl.when(c == 0)
        def _():
            run(lidx_hbm, hl_hbm)

        @pl.when(c == 1)
        def _():
            run(ridx_hbm, hr_hbm)

    dt = h_node.dtype
    call = pl.kernel(
        body,
        out_type=[jax.ShapeDtypeStruct((E, D), dt),
                  jax.ShapeDtypeStruct((E, D), dt)],
        mesh=mesh,
        scratch_types=[
            pltpu.VMEM((nch_tile, CH), jnp.int32),
            pltpu.VMEM((NB, CH, D), dt),
            pltpu.SemaphoreType.DMA,
            pltpu.SemaphoreType.DMA,
        ] + [pltpu.VMEM((CH,), jnp.int32) for _ in range(NB)],
    )
    return call(h_node, lidx, ridx)


def _scatter_regather(msgs, idxs, lidx, ridx, p_l, p_r):
    (mlA, mrA, mlB, mrB) = msgs
    (lidxA, ridxA, lidxB, ridxB) = idxs
    E = lidx.shape[0] * lidx.shape[1] * lidx.shape[2]
    D = mlA.shape[1]
    n_pad = p_l.shape[0]
    nch_tile = lidx.shape[1]
    nch_a = lidxA.shape[1]
    nch_b = lidxB.shape[1]
    rows_per_tile = n_pad // NS
    mesh = plsc.VectorSubcoreMesh(core_axis_name="c", subcore_axis_name="s")

    def body(mlA_hbm, mrA_hbm, mlB_hbm, mrB_hbm, lidxA_hbm, ridxA_hbm,
             lidxB_hbm, ridxB_hbm, lidx_hbm, ridx_hbm, pl_hbm, pr_hbm,
             gl_hbm, gr_hbm, accum, idx_v, buf_v, gsem, wsem, *idxb):
        c = lax.axis_index("c")
        s = lax.axis_index("s")

        def init(p_hbm):
            def seed(j, carry):
                r0 = s * rows_per_tile + j * CH
                pltpu.sync_copy(p_hbm.at[pl.ds(r0, CH)], buf_v.at[0])
                pltpu.sync_copy(buf_v.at[0], accum.at[pl.ds(r0, CH)])
                return carry

            lax.fori_loop(0, rows_per_tile // CH, seed, 0)

        def run_scatter(m_hbm, sidx_hbm, nch):
            pltpu.sync_copy(sidx_hbm.at[s], idx_v.at[pl.ds(0, nch)])

            def do_group(k0, nb):
                mds = []
                for b in range(nb):
                    mds.append(pltpu.async_copy(
                        m_hbm.at[pl.ds((s * nch + k0 + b) * CH, CH)],
                        buf_v.at[b], gsem))
                for b in range(nb):
                    _stage_idx(idx_v, k0 + b, idxb[b])
                    mds[b].wait()
                    pltpu.sync_copy(buf_v.at[b], accum.at[idxb[b]], add=True)

            _grouped(nch, NB3, do_group)

        def run_gather(gidx_hbm, out_hbm):
            pltpu.sync_copy(gidx_hbm.at[s], idx_v)

            def do_group(k0, nb):
                wds = []
                for b in range(nb):
                    _stage_idx(idx_v, k0 + b, idxb[b])
                    pltpu.async_copy(accum.at[idxb[b]], buf_v.at[b],
                                     gsem).wait()
                    wds.append(pltpu.async_copy(
                        buf_v.at[b],
                        out_hbm.at[pl.ds((s * nch_tile + k0 + b) * CH, CH)],
                        wsem))
                for d in wds:
                    d.wait()

            _grouped(nch_tile, NB3, do_group)

        @pl.when(c == 0)
        def _():
            init(pl_hbm)

        @pl.when(c == 1)
        def _():
            init(pr_hbm)

        plsc.subcore_barrier()

        @pl.when(c == 0)
        def _():
            run_scatter(mlA_hbm, ridxA_hbm, nch_a)
            run_scatter(mlB_hbm, ridxB_hbm, nch_b)

        @pl.when(c == 1)
        def _():
            run_scatter(mrA_hbm, lidxA_hbm, nch_a)
            run_scatter(mrB_hbm, lidxB_hbm, nch_b)

        plsc.subcore_barrier()

        @pl.when(c == 0)
        def _():
            run_gather(lidx_hbm, gl_hbm)

        @pl.when(c == 1)
        def _():
            run_gather(ridx_hbm, gr_hbm)

    call = pl.kernel(
        body,
        out_type=[jax.ShapeDtypeStruct((E, D), F32),
                  jax.ShapeDtypeStruct((E, D), F32)],
        mesh=mesh,
        scratch_types=[
            pltpu.VMEM_SHARED((n_pad, D), F32),
            pltpu.VMEM((nch_tile, CH), jnp.int32),
            pltpu.VMEM((NB3, CH, D), F32),
            pltpu.SemaphoreType.DMA,
            pltpu.SemaphoreType.DMA,
        ] + [pltpu.VMEM((CH,), jnp.int32) for _ in range(NB3)],
    )
    return call(mlA, mrA, mlB, mrB, lidxA, ridxA, lidxB, ridxB,
                lidx, ridx, p_l, p_r)


def _node_proj(h_node, nl_W, nr_W, n_pad):
    N, D = h_node.shape
    tile_n = n_pad // NS

    def body(hn_ref, nl_ref, nr_ref, pl_ref, pr_ref):
        hn = hn_ref[...]
        pl_ref[...] = jnp.dot(hn, nl_ref[...], preferred_element_type=F32)
        pr_ref[...] = jnp.dot(hn, nr_ref[...], preferred_element_type=F32)

    node_spec = pl.BlockSpec((tile_n, D), lambda i: (i, 0))
    wspec = pl.BlockSpec((D, D), lambda i: (0, 0))
    return pl.pallas_call(
        body,
        grid=(n_pad // tile_n,),
        in_specs=[node_spec, wspec, wspec],
        out_specs=[node_spec, node_spec],
        out_shape=[jax.ShapeDtypeStruct((n_pad, D), F32)] * 2,
    )(h_node, nl_W, nr_W)


def _edge_mlp(xb, xl, xr, t, Wb, Wn, W1, b1, W2, b2, Wgb, Wgn, wgt, bg1,
              Wg2, bg2, *, tile_e, e_off=0):
    E, D = xl.shape
    for cand_t in (tile_e, 3200, 1280, 640, 320, 160, E):
        if E % cand_t == 0 and e_off % cand_t == 0:
            tile_e = cand_t
            break
    off_t = e_off // tile_e

    def body(xb_ref, xl_ref, xr_ref, t_ref, Wb_ref, Wn_ref, W1_ref, b1_ref,
             W2_ref, b2_ref, Wgb_ref, Wgn_ref, wgt_ref, bg1_ref, Wg2_ref,
             bg2_ref, al_ref, ar_ref):
        b = xb_ref[...]
        tcol = t_ref[...]
        xn = (xl_ref, xr_ref)
        outs = (al_ref, ar_ref)
        for p in range(2):
            x = xn[p][...].astype(F32)
            inter = (jnp.dot(b, Wb_ref[p], preferred_element_type=F32) *
                     jnp.dot(x, Wn_ref[p], preferred_element_type=F32))
            inter = jnp.maximum(
                jnp.dot(inter, W1_ref[p], preferred_element_type=F32)
                + b1_ref[p], 0.0)
            inter = (jnp.dot(inter, W2_ref[p], preferred_element_type=F32)
                     + b2_ref[p])
            g = (jnp.dot(b, Wgb_ref[p], preferred_element_type=F32)
                 + jnp.dot(x, Wgn_ref[p], preferred_element_type=F32)
                 + tcol * wgt_ref[p] + bg1_ref[p])
            g = (jnp.dot(jnp.maximum(g, 0.0), Wg2_ref[p],
                         preferred_element_type=F32) + bg2_ref[p])
            outs[p][...] = inter * jax.nn.sigmoid(g)

    grid = (E // tile_e,)
    edge_spec = pl.BlockSpec((tile_e, D), lambda i: (i, 0))
    off_spec = pl.BlockSpec((tile_e, D), lambda i: (i + off_t, 0))
    t_spec = pl.BlockSpec((tile_e, 1), lambda i: (i + off_t, 0))

    def wspec(a):
        nd = a.ndim
        return pl.BlockSpec(a.shape, lambda i, _n=nd: (0,) * _n)

    weights = (Wb, Wn, W1, b1, W2, b2, Wgb, Wgn, wgt, bg1, Wg2, bg2)
    return pl.pallas_call(
        body,
        grid=grid,
        in_specs=[off_spec, edge_spec, edge_spec, t_spec]
                 + [wspec(w) for w in weights],
        out_specs=[edge_spec, edge_spec],
        out_shape=[jax.ShapeDtypeStruct((E, D), F32)] * 2,
    )(xb, xl, xr, t, *weights)


def _final(gl, gr, xb, sf_W, dbias, ln_g, ln_b, ot_W, ot_b, *, tile_e):
    E, D = gl.shape

    def body(gl_ref, gr_ref, xb_ref, sfW_ref, dbias_ref, lng_ref, lnb_ref,
             otW_ref, otb_ref, out_ref):
        h = (gl_ref[...] + gr_ref[...]
             + jnp.dot(xb_ref[...], sfW_ref[...], preferred_element_type=F32)
             + dbias_ref[...])
        mu = jnp.mean(h, axis=-1, keepdims=True)
        var = jnp.mean((h - mu) ** 2, axis=-1, keepdims=True)
        h = (h - mu) * lax.rsqrt(var + 1e-5) * lng_ref[...] + lnb_ref[...]
        out_ref[...] = (jnp.dot(jnp.maximum(h, 0.0), otW_ref[...],
                                preferred_element_type=F32) + otb_ref[...])

    grid = (E // tile_e,)
    edge_spec = pl.BlockSpec((tile_e, D), lambda i: (i, 0))

    def wspec(a):
        nd = a.ndim
        return pl.BlockSpec(a.shape, lambda i, _n=nd: (0,) * _n)

    return pl.pallas_call(
        body,
        grid=grid,
        in_specs=[edge_spec, edge_spec, edge_spec, wspec(sf_W), wspec(dbias),
                  wspec(ln_g), wspec(ln_b), wspec(ot_W), wspec(ot_b)],
        out_specs=edge_spec,
        out_shape=jax.ShapeDtypeStruct((E, D), F32),
    )(gl, gr, xb, sf_W, dbias, ln_g, ln_b, ot_W, ot_b)


def kernel(h_bond, bond_index, h_node, bond_time,
           L_Wb, L_Wn, L_W1, L_b1, L_W2, L_b2, L_Wg1, L_bg1, L_Wg2, L_bg2,
           R_Wb, R_Wn, R_W1, R_b1, R_W2, R_b2, R_Wg1, R_bg1, R_Wg2, R_bg2,
           nl_W, nl_b, nr_W, nr_b, sf_W, sf_b, ln_g, ln_b, ot_W, ot_b):
    E, D = h_bond.shape
    N = h_node.shape[0]
    left, right = bond_index[0], bond_index[1]
    lidx = left.reshape(NS, E // CH // NS, CH)
    ridx = right.reshape(NS, E // CH // NS, CH)

    EA = (E * 2 // 5) // (NS * CH) * (NS * CH)
    rsh = lambda a: a.reshape(NS, -1, CH)
    lidxA, ridxA = rsh(left[:EA]), rsh(right[:EA])
    lidxB, ridxB = rsh(left[EA:]), rsh(right[EA:])

    hlA, hrA = _pair_gather(h_node, lidxA, ridxA)
    hlB, hrB = _pair_gather(h_node, lidxB, ridxB)

    n_pad = ((N + NS * CH - 1) // (NS * CH)) * (NS * CH)
    p_l, p_r = _node_proj(h_node, nl_W, nr_W, n_pad)

    stk = lambda a, b: jnp.stack([a, b])
    Wb = stk(L_Wb, R_Wb)
    Wn = stk(L_Wn, R_Wn)
    W1 = stk(L_W1, R_W1)
    b1 = stk(L_b1.reshape(1, -1), R_b1.reshape(1, -1))
    W2 = stk(L_W2, R_W2)
    b2 = stk(L_b2.reshape(1, -1), R_b2.reshape(1, -1))
    Wgb = stk(L_Wg1[:D], R_Wg1[:D])
    Wgn = stk(L_Wg1[D:2 * D], R_Wg1[D:2 * D])
    wgt = stk(L_Wg1[2 * D:], R_Wg1[2 * D:])
    bg1 = stk(L_bg1.reshape(1, -1), R_bg1.reshape(1, -1))
    Wg2 = stk(L_Wg2, R_Wg2)
    bg2 = stk(L_bg2.reshape(1, -1), R_bg2.reshape(1, -1))
    dbias = (nl_b + nr_b + sf_b).reshape(1, D)

    mlA, mrA = _edge_mlp(
        h_bond, hlA, hrA, bond_time, Wb, Wn, W1, b1, W2, b2,
        Wgb, Wgn, wgt, bg1, Wg2, bg2, tile_e=6400)
    mlB, mrB = _edge_mlp(
        h_bond, hlB, hrB, bond_time, Wb, Wn, W1, b1, W2, b2,
        Wgb, Wgn, wgt, bg1, Wg2, bg2, tile_e=6400, e_off=EA)

    gl, gr = _scatter_regather(
        (mlA, mrA, mlB, mrB), (lidxA, ridxA, lidxB, ridxB),
        lidx, ridx, p_l, p_r)

    return _final(gl, gr, h_bond, sf_W, dbias, ln_g.reshape(1, D),
                  ln_b.reshape(1, D), ot_W, ot_b.reshape(1, D), tile_e=6400)
